# Initial kernel scaffold; baseline (speedup 1.0000x reference)
import jax
import jax.numpy as jnp
from jax.experimental import pallas as pl


def kernel(lane_dist, same_obstacle_mask, ts_mask):
    raise NotImplementedError("write your pallas kernel here")


# speedup vs baseline: 5.4501x; 5.4501x over previous
import jax
import jax.numpy as jnp
from jax.experimental import pallas as pl
from jax.experimental.pallas import tpu as pltpu

M_TOTAL = 2097152
NBLK = 16
BLK = M_TOTAL // NBLK


def _seg_run_min(d, seg, reverse):
    v = d
    n = d.shape[0]
    k = 1
    while k < n:
        if not reverse:
            sv = jnp.concatenate([jnp.full((k,), jnp.inf, v.dtype), v[: n - k]])
            ss = jnp.concatenate([jnp.full((k,), -1, seg.dtype), seg[: n - k]])
        else:
            sv = jnp.concatenate([v[k:], jnp.full((k,), jnp.inf, v.dtype)])
            ss = jnp.concatenate([seg[k:], jnp.full((k,), -1, seg.dtype)])
        v = jnp.where(ss == seg, jnp.minimum(v, sv), v)
        k *= 2
    return v


def _fwd_kernel(d_ref, seg_ref, condA_ref, cmin_ref, cseg_ref):
    b = pl.program_id(0)

    @pl.when(b == 0)
    def _init():
        cmin_ref[0] = jnp.inf
        cseg_ref[0] = -1

    d = d_ref[...]
    seg = seg_ref[...]
    carry_min = cmin_ref[0]
    carry_seg = cseg_ref[0]

    v = _seg_run_min(d, seg, reverse=False)
    v = jnp.where(seg == carry_seg, jnp.minimum(v, carry_min), v)
    pv = jnp.concatenate([jnp.full((1,), carry_min, d.dtype), v[:-1]])
    ps = jnp.concatenate([jnp.full((1,), carry_seg, seg.dtype), seg[:-1]])
    pe = jnp.where(ps == seg, pv, jnp.inf)
    condA_ref[...] = (d < pe).astype(jnp.float32)

    s_last = jnp.max(seg)
    tmin = jnp.min(jnp.where(seg == s_last, d, jnp.inf))
    cmin_ref[0] = jnp.where(s_last == carry_seg, jnp.minimum(tmin, carry_min), tmin)
    cseg_ref[0] = s_last


def _bwd_kernel(d_ref, seg_ref, condA_ref, tspos_ref, out_ref, cmin_ref, cseg_ref):
    b = pl.program_id(0)

    @pl.when(b == 0)
    def _init():
        cmin_ref[0] = jnp.inf
        cseg_ref[0] = -1

    d = d_ref[...]
    seg = seg_ref[...]
    carry_min = cmin_ref[0]
    carry_seg = cseg_ref[0]

    v = _seg_run_min(d, seg, reverse=True)
    v = jnp.where(seg == carry_seg, jnp.minimum(v, carry_min), v)
    sv = jnp.concatenate([v[1:], jnp.full((1,), carry_min, d.dtype)])
    ss = jnp.concatenate([seg[1:], jnp.full((1,), carry_seg, seg.dtype)])
    se = jnp.where(ss == seg, sv, jnp.inf)
    condB = d <= se

    out_ref[...] = (
        (condA_ref[...] > 0.0) & condB & (tspos_ref[...] > 0.0)
    ).astype(jnp.float32)

    s_first = jnp.min(seg)
    tmin = jnp.min(jnp.where(seg == s_first, d, jnp.inf))
    cmin_ref[0] = jnp.where(s_first == carry_seg, jnp.minimum(tmin, carry_min), tmin)
    cseg_ref[0] = s_first


def _fwd_spec():
    return pl.BlockSpec((BLK,), lambda i: (i,))


def _bwd_spec():
    return pl.BlockSpec((BLK,), lambda i: (NBLK - 1 - i,))


@jax.jit
def kernel(lane_dist, same_obstacle_mask, ts_mask):
    seg = same_obstacle_mask.reshape(-1).astype(jnp.int32)
    tspos = ts_mask[seg].astype(jnp.float32)

    condA = pl.pallas_call(
        _fwd_kernel,
        grid=(NBLK,),
        in_specs=[_fwd_spec(), _fwd_spec()],
        out_specs=_fwd_spec(),
        out_shape=jax.ShapeDtypeStruct((M_TOTAL,), jnp.float32),
        scratch_shapes=[
            pltpu.SMEM((1,), jnp.float32),
            pltpu.SMEM((1,), jnp.int32),
        ],
    )(lane_dist, seg)

    outf = pl.pallas_call(
        _bwd_kernel,
        grid=(NBLK,),
        in_specs=[_bwd_spec(), _bwd_spec(), _bwd_spec(), _bwd_spec()],
        out_specs=_bwd_spec(),
        out_shape=jax.ShapeDtypeStruct((M_TOTAL,), jnp.float32),
        scratch_shapes=[
            pltpu.SMEM((1,), jnp.float32),
            pltpu.SMEM((1,), jnp.int32),
        ],
    )(lane_dist, seg, condA, tspos)

    return outf.astype(jnp.bool_)
